# baseline (speedup 1.0000x reference)
import functools

import jax
import jax.numpy as jnp
from jax import lax
from jax.experimental import pallas as pl
from jax.experimental.pallas import tpu as pltpu
from jax.experimental.pallas import tpu_sc as plsc

N_CORES = 2
N_SUB = 16

N_PHASE = 3
PH = 3336
OWN = 209
CAP = 10240
CH = 80
N_CHUNKS = CAP // CH


def _make_sc_agg(n_nodes, fh):
    accr = PH + 8
    rps = (PH // N_SUB) // 8 * 8
    zlast = accr - 15 * rps
    mesh = plsc.VectorSubcoreMesh(core_axis_name="c", subcore_axis_name="s")
    phase_rows = []
    for p in range(N_PHASE):
        valid = min(PH, n_nodes - p * PH)
        phase_rows.append(valid - 15 * rps)

    @functools.partial(
        pl.kernel,
        mesh=mesh,
        out_type=jax.ShapeDtypeStruct((2 * n_nodes, fh), jnp.float32),
        scratch_types=[
            pltpu.VMEM((N_CHUNKS, CH), jnp.int32),
            pltpu.VMEM((N_CHUNKS, CH), jnp.int32),
            pltpu.VMEM((CH, fh), jnp.float32),
            pltpu.VMEM((CH, fh), jnp.float32),
            pltpu.VMEM_SHARED((accr, fh), jnp.float32),
            pltpu.SemaphoreType.DMA,
            pltpu.SemaphoreType.DMA,
        ],
        name="sc_edge_segment_sum",
    )
    def sc_agg(table, src_r, dst_r, zeros, out, idxb, dstb, buf_a, buf_b,
               acc, sem_a, sem_b):
        c = lax.axis_index("c")
        s = lax.axis_index("s")

        for p in range(N_PHASE):
            @pl.when(s < N_SUB - 1)
            def _():
                pltpu.sync_copy(zeros.at[pl.ds(0, rps)],
                                acc.at[pl.ds(s * rps, rps)])

            @pl.when(s == N_SUB - 1)
            def _():
                pltpu.sync_copy(zeros.at[pl.ds(0, zlast)],
                                acc.at[pl.ds((N_SUB - 1) * rps, zlast)])

            pltpu.sync_copy(src_r.at[c, p, s], idxb)
            pltpu.sync_copy(dst_r.at[p, s], dstb)
            plsc.subcore_barrier()

            pltpu.async_copy(table.at[idxb.at[0]], buf_a, sem_a)
            pltpu.async_copy(table.at[idxb.at[1]], buf_b, sem_b)

            def body(k2, carry):
                ka = 2 * k2
                kb = 2 * k2 + 1
                pltpu.make_async_copy(table.at[idxb.at[ka]], buf_a,
                                      sem_a).wait()
                pltpu.sync_copy(buf_a, acc.at[dstb.at[ka]], add=True)

                @pl.when(ka + 2 < N_CHUNKS)
                def _():
                    pltpu.async_copy(table.at[idxb.at[ka + 2]], buf_a, sem_a)

                pltpu.make_async_copy(table.at[idxb.at[kb]], buf_b,
                                      sem_b).wait()
                pltpu.sync_copy(buf_b, acc.at[dstb.at[kb]], add=True)

                @pl.when(kb + 2 < N_CHUNKS)
                def _():
                    pltpu.async_copy(table.at[idxb.at[kb + 2]], buf_b, sem_b)

                return carry

            lax.fori_loop(0, N_CHUNKS // 2, body, 0)
            plsc.subcore_barrier()

            obase = c * n_nodes + p * PH
            rl = phase_rows[p]

            @pl.when(s < N_SUB - 1)
            def _():
                pltpu.sync_copy(acc.at[pl.ds(s * rps, rps)],
                                out.at[pl.ds(obase + s * rps, rps)])

            @pl.when(s == N_SUB - 1)
            def _():
                pltpu.sync_copy(
                    acc.at[pl.ds((N_SUB - 1) * rps, rl)],
                    out.at[pl.ds(obase + (N_SUB - 1) * rps, rl)])

    return sc_agg


def _tc_dot(lhs, w, b, relu, rows):
    n, k = lhs.shape
    m = w.shape[1]
    grid = n // rows

    def body(l_ref, w_ref, b_ref, o_ref):
        t = jnp.dot(l_ref[...], w_ref[...],
                    preferred_element_type=jnp.float32) + b_ref[...]
        o_ref[...] = jnp.maximum(t, 0.0) if relu else t

    return pl.pallas_call(
        body,
        grid=(grid,),
        in_specs=[pl.BlockSpec((rows, k), lambda i: (i, 0)),
                  pl.BlockSpec((k, m), lambda i: (0, 0)),
                  pl.BlockSpec((1, m), lambda i: (0, 0))],
        out_specs=pl.BlockSpec((rows, m), lambda i: (i, 0)),
        out_shape=jax.ShapeDtypeStruct((n, m), jnp.float32),
        name="tc_dot",
    )(lhs, w, b)


def _tc_add1(x, ragg3):
    n, f = x.shape
    rows = 1000

    def body(x_ref, r_ref, o_ref):
        o_ref[...] = x_ref[...] + r_ref[0]

    return pl.pallas_call(
        body,
        grid=(n // rows,),
        in_specs=[pl.BlockSpec((rows, f), lambda i: (i, 0)),
                  pl.BlockSpec((2, rows, f), lambda i: (0, i, 0))],
        out_specs=pl.BlockSpec((rows, f), lambda i: (i, 0)),
        out_shape=jax.ShapeDtypeStruct((n, f), jnp.float32),
        name="tc_add1",
    )(x, ragg3)


def _tc_addg(x, ragg3):
    n, f = x.shape
    fh = f // 2
    rows = 1000

    def body(x_ref, r_ref, o_ref):
        o_ref[...] = x_ref[...] + jnp.concatenate([r_ref[0], r_ref[1]],
                                                  axis=1)

    return pl.pallas_call(
        body,
        grid=(n // rows,),
        in_specs=[pl.BlockSpec((rows, f), lambda i: (i, 0)),
                  pl.BlockSpec((2, rows, fh), lambda i: (0, i, 0))],
        out_specs=pl.BlockSpec((rows, f), lambda i: (i, 0)),
        out_shape=jax.ShapeDtypeStruct((n, f), jnp.float32),
        name="tc_addg",
    )(x, ragg3)


def _tc_split(x):
    n, f = x.shape
    fh = f // 2
    rows = 1000

    def body(x_ref, t_ref):
        t_ref[0] = x_ref[:, :fh]
        t_ref[1] = x_ref[:, fh:]

    return pl.pallas_call(
        body,
        grid=(n // rows,),
        in_specs=[pl.BlockSpec((rows, f), lambda i: (i, 0))],
        out_specs=pl.BlockSpec((2, rows, fh), lambda i: (0, i, 0)),
        out_shape=jax.ShapeDtypeStruct((2, n, fh), jnp.float32),
        name="tc_split",
    )(x)


def _tc_pool(x, batch2, bp):
    n, f = x.shape
    rows = 1000

    def body(x_ref, b_ref, o_ref, pool_s):
        i = pl.program_id(0)

        @pl.when(i == 0)
        def _():
            pool_s[...] = jnp.zeros_like(pool_s)

        ids = lax.broadcasted_iota(jnp.int32, (rows, bp), 1)
        onehot = (b_ref[...] == ids).astype(jnp.float32)
        pool_s[...] += lax.dot_general(
            onehot, x_ref[...], (((0,), (0,)), ((), ())),
            preferred_element_type=jnp.float32,
            precision=lax.Precision.HIGHEST)

        @pl.when(i == pl.num_programs(0) - 1)
        def _():
            o_ref[...] = pool_s[...]

    return pl.pallas_call(
        body,
        grid=(n // rows,),
        in_specs=[pl.BlockSpec((rows, f), lambda i: (i, 0)),
                  pl.BlockSpec((rows, 1), lambda i: (i, 0))],
        out_specs=pl.BlockSpec((bp, f), lambda i: (0, 0)),
        out_shape=jax.ShapeDtypeStruct((bp, f), jnp.float32),
        scratch_shapes=[pltpu.VMEM((bp, f), jnp.float32)],
        name="tc_pool",
    )(x, batch2)


def _bucketize(src, dst, n):
    e = src.shape[0]
    p = dst // PH
    local = dst - p * PH
    s = local // OWN
    key = (p * N_SUB + s).astype(jnp.int32)
    order = jnp.argsort(key, stable=True)
    skey = key[order]
    start = jnp.searchsorted(skey,
                             jnp.arange(N_PHASE * N_SUB, dtype=skey.dtype))
    rank = jnp.arange(e, dtype=jnp.int32) - start[skey].astype(jnp.int32)
    slot = skey * CAP + rank
    slot = jnp.where(rank < CAP, slot, 2 ** 30)
    tot = N_PHASE * N_SUB * CAP
    src_pad = jnp.zeros((tot,), jnp.int32).at[slot].set(
        src[order], mode="drop")
    dst_pad = jnp.full((tot,), PH, jnp.int32).at[slot].set(
        local[order], mode="drop")
    src_r = jnp.stack([src_pad, src_pad + n]).reshape(
        2, N_PHASE, N_SUB, N_CHUNKS, CH)
    dst_r = dst_pad.reshape(N_PHASE, N_SUB, N_CHUNKS, CH)
    return src_r, dst_r


def kernel(x, edge_index, batch,
           l1_W1, l1_b1, l1_W2, l1_b2, l1_g, l1_be,
           l2_W1, l2_b1, l2_W2, l2_b2, l2_g, l2_be,
           l3_W1, l3_b1, l3_W2, l3_b2, l3_g, l3_be,
           l4_W1, l4_b1, l4_W2, l4_b2, l4_g, l4_be,
           l5_W1, l5_b1, l5_W2, l5_b2, l5_g, l5_be,
           l6_W1, l6_b1, l6_W2, l6_b2, l6_g, l6_be,
           fcxd_W, fcxd_b, fc1_W, fc1_b, fc2_W, fc2_b, out_W, out_b):
    n, f_in = x.shape
    dim = l1_W1.shape[1]
    fh = dim // 2
    n_graphs = 300
    bp = 384

    src = edge_index[0]
    dst = edge_index[1]
    src_r, dst_r = _bucketize(src, dst, n)
    zeros_t = jnp.zeros((PH + 8 - 15 * ((PH // N_SUB) // 8 * 8), fh),
                        jnp.float32)

    sc_agg_a = _make_sc_agg(n, fh)
    sc_agg_b = _make_sc_agg(n, fh)

    tab1 = jnp.concatenate([x, x], axis=0)
    ragg3 = sc_agg_a(tab1, src_r, dst_r, zeros_t).reshape(2, n, fh)
    z = _tc_add1(x, ragg3)
    y1 = _tc_dot(z, l1_W1, l1_b1.reshape(1, -1), True, 1000)
    h = _tc_dot(y1, l1_W2, l1_b2.reshape(1, -1), True, 1000)
    mu = jnp.mean(h, axis=0)
    var = jnp.var(h, axis=0)
    xx = l1_g * (h - mu) / jnp.sqrt(var + 1e-5) + l1_be

    w1s = jnp.stack([l2_W1, l3_W1, l4_W1, l5_W1, l6_W1])
    b1s = jnp.stack([l2_b1, l3_b1, l4_b1, l5_b1, l6_b1])
    w2s = jnp.stack([l2_W2, l3_W2, l4_W2, l5_W2, l6_W2])
    b2s = jnp.stack([l2_b2, l3_b2, l4_b2, l5_b2, l6_b2])
    gs = jnp.stack([l2_g, l3_g, l4_g, l5_g, l6_g])
    bes = jnp.stack([l2_be, l3_be, l4_be, l5_be, l6_be])

    def step(carry, ws):
        xx = carry
        w1, b1, w2, b2, g, be = ws
        tab3 = _tc_split(xx)
        ragg3 = sc_agg_b(tab3.reshape(2 * n, fh), src_r, dst_r,
                         zeros_t).reshape(2, n, fh)
        z = _tc_addg(xx, ragg3)
        y1 = _tc_dot(z, w1, b1.reshape(1, -1), True, 1000)
        h = _tc_dot(y1, w2, b2.reshape(1, -1), True, 1000)
        mu = jnp.mean(h, axis=0)
        var = jnp.var(h, axis=0)
        return g * (h - mu) / jnp.sqrt(var + 1e-5) + be, 0.0

    xx, _ = lax.scan(step, xx, (w1s, b1s, w2s, b2s, gs, bes))

    pooled = _tc_pool(xx, batch.reshape(n, 1), bp)
    t = _tc_dot(pooled, fcxd_W, fcxd_b.reshape(1, -1), True, bp)
    t = _tc_dot(t, fc1_W, fc1_b.reshape(1, -1), True, bp)
    t = _tc_dot(t, fc2_W, fc2_b.reshape(1, -1), True, bp)
    o = _tc_dot(t, out_W, out_b.reshape(1, -1), False, bp)
    return o[:n_graphs, 0]
